# baseline (speedup 1.0000x reference)
import jax
import jax.numpy as jnp
from jax import lax
from jax.experimental import pallas as pl
from jax.experimental.pallas import tpu as pltpu
from jax.experimental.pallas import tpu_sc as plsc

N = 10000
E = 320000
C = 64
G = 128
IN_DIM = 4

_COMBOS = [(i, f, o) for i in range(3) for f in range(3)
           for o in range(abs(f - i), min(i + f + 1, 3))]

NC = 2
NS = 16
NW = NC * NS
CH = 64
EP = 331776
CPW = EP // (CH * NW)
NRING = 3
NIR = 6

ACC_N = N
RPS = 632
RPS_LAST = ACC_N - 15 * RPS

EB = 2048
NBLK = 1000


def _ssp(x):
    return jnp.maximum(x, 0.0) + jnp.log1p(jnp.exp(-jnp.abs(x))) - jnp.log(2.0)


def _elu(x):
    return jnp.where(x > 0, x, jnp.exp(jnp.minimum(x, 0.0)) - 1.0)


def _make_edgew_body(ng):
    def _edgew_body(ea_ref, a_ref, amu_ref, w1_ref, b1_ref, bd_ref, b2_ref,
                    w_ref):
        ea = ea_ref[...]
        r = jnp.sqrt(jnp.sum(ea * ea, axis=1, keepdims=True) + 1e-12)
        sr = r * a_ref[0, 0]
        d = sr - amu_ref[...]
        rb = jnp.exp(-(d * d))
        a = jnp.maximum(
            jnp.dot(rb, w1_ref[...], preferred_element_type=jnp.float32)
            + b1_ref[...], 0.0)
        eb = pl.program_id(0)
        ids = eb * EB + lax.broadcasted_iota(jnp.int32, (EB, 1), 0)
        valid = ids < E
        for g in range(ng):
            w = (jnp.dot(a, bd_ref[:, G * g:G * (g + 1)],
                         preferred_element_type=jnp.float32) + b2_ref[g])
            w_ref[g] = jnp.where(valid, w, 0.0)
    return _edgew_body


def _edge_weights(ng, nb, ea_pad, aS, amuS, w1S, b1S, bdS, b2S):
    nblocks = EP // EB
    return pl.pallas_call(
        _make_edgew_body(ng),
        grid=(nblocks,),
        in_specs=[
            pl.BlockSpec((EB, 3), lambda eb: (eb, 0)),
            pl.BlockSpec((1, 1), lambda eb: (0, 0)),
            pl.BlockSpec((1, nb), lambda eb: (0, 0)),
            pl.BlockSpec((nb, 24 * ng), lambda eb: (0, 0)),
            pl.BlockSpec((1, 24 * ng), lambda eb: (0, 0)),
            pl.BlockSpec((24 * ng, G * ng), lambda eb: (0, 0)),
            pl.BlockSpec((ng, 1, G), lambda eb: (0, 0, 0)),
        ],
        out_specs=pl.BlockSpec((ng, EB, G), lambda eb: (0, eb, 0)),
        out_shape=jax.ShapeDtypeStruct((ng, EP, G), jnp.float32),
    )(ea_pad, aS, amuS, w1S, b1S, bdS, b2S)


def _nodeh1_body(x_ref, lin1_ref, win_ref, h_ref):
    out = jnp.dot(x_ref[...], lin1_ref[...], preferred_element_type=jnp.float32)
    for g in range(2):
        h_ref[g] = jnp.dot(out, win_ref[g], preferred_element_type=jnp.float32)


def _node_h1(x, lin1, wincat):
    return pl.pallas_call(
        _nodeh1_body,
        grid=(N // NBLK,),
        in_specs=[
            pl.BlockSpec((NBLK, IN_DIM), lambda nb: (nb, 0)),
            pl.BlockSpec((IN_DIM, C), lambda nb: (0, 0)),
            pl.BlockSpec((2, C, G), lambda nb: (0, 0, 0)),
        ],
        out_specs=pl.BlockSpec((2, NBLK, G), lambda nb: (0, nb, 0)),
        out_shape=jax.ShapeDtypeStruct((2, N, G), jnp.float32),
    )(x, lin1, wincat)


def _make_sc_body(grp):
    def _sc_body(htab, wtab, idxpk, zero_hbm, out_hbm,
                 idxb, rows, wv, acc_sh, gsem, wsem, ssem, isem):
        c = lax.axis_index("c")
        s = lax.axis_index("s")
        wid = s * NC + c
        r0 = s * RPS

        def group_body(g, carry):
            @pl.when(s < NS - 1)
            def _zero_main():
                pltpu.sync_copy(zero_hbm.at[pl.ds(r0, RPS)],
                                acc_sh.at[pl.ds(r0, RPS)])

            @pl.when(s == NS - 1)
            def _zero_tail():
                pltpu.sync_copy(zero_hbm.at[pl.ds(15 * RPS, RPS_LAST)],
                                acc_sh.at[pl.ds(15 * RPS, RPS_LAST)])
            plsc.subcore_barrier()
            off = g * N

            def issue_idx(ch):
                ib = ch % NIR
                pltpu.async_copy(idxpk.at[wid * CPW + ch], idxb.at[ib],
                                 isem.at[ib])

            def issue(ch, b):
                ib = ch % NIR
                gc = wid * CPW + ch
                pltpu.make_async_copy(idxpk.at[gc], idxb.at[ib],
                                      isem.at[ib]).wait()
                for j in range(CH // 16):
                    sl = pl.ds(j * 16, 16)
                    idxb[ib, 0, sl] = idxb[ib, 0, sl] + off
                pltpu.async_copy(htab.at[idxb.at[ib, 0]], rows.at[b],
                                 gsem.at[b])
                pltpu.async_copy(wtab.at[pl.ds(g * EP + gc * CH, CH)],
                                 wv.at[b], wsem.at[b])

            for p in range(4):
                issue_idx(p)
            issue(0, 0)
            issue(1, 1)

            def outer(io, carry2):
                for b in range(NRING):
                    ch = io * NRING + b
                    gc = wid * CPW + ch
                    ib = ch % NIR
                    pltpu.make_async_copy(htab.at[idxb.at[ib, 0]],
                                          rows.at[b], gsem.at[b]).wait()
                    pltpu.make_async_copy(wtab.at[pl.ds(g * EP + gc * CH, CH)],
                                          wv.at[b], wsem.at[b]).wait()

                    rb = rows.at[b]
                    wb = wv.at[b]

                    @plsc.parallel_loop(0, CH, step=1, unroll=8)
                    def mul_body(k):
                        for cc in range(G // 16):
                            sl = pl.ds(cc * 16, 16)
                            rb[k, sl] = rb[k, sl] * wb[k, sl]

                    bp = (b + 2) % NRING

                    @pl.when(ch >= 1)
                    def _wait_prev_scatter():
                        pltpu.make_async_copy(
                            rows.at[bp], acc_sh.at[idxb.at[(ch + 5) % NIR, 1]],
                            ssem.at[bp]).wait()
                    pltpu.async_copy(rows.at[b], acc_sh.at[idxb.at[ib, 1]],
                                     ssem.at[b], add=True)

                    @pl.when(ch + 2 < CPW)
                    def _issue_ahead():
                        issue(ch + 2, (b + 2) % NRING)

                    @pl.when(ch + 4 < CPW)
                    def _issue_idx_ahead():
                        issue_idx(ch + 4)
                return carry2
            lax.fori_loop(0, CPW // NRING, outer, 0)

            bl = (CPW - 1) % NRING
            pltpu.make_async_copy(rows.at[bl],
                                  acc_sh.at[idxb.at[(CPW - 1) % NIR, 1]],
                                  ssem.at[bl]).wait()
            plsc.subcore_barrier()

            @pl.when(s < NS - 1)
            def _dump_main():
                pltpu.sync_copy(acc_sh.at[pl.ds(r0, RPS)],
                                out_hbm.at[g].at[c].at[pl.ds(r0, RPS)])

            @pl.when(s == NS - 1)
            def _dump_tail():
                pltpu.sync_copy(
                    acc_sh.at[pl.ds(15 * RPS, RPS_LAST)],
                    out_hbm.at[g].at[c].at[pl.ds(15 * RPS, RPS_LAST)])
            plsc.subcore_barrier()
            return carry
        lax.fori_loop(0, grp, group_body, 0)
    return _sc_body


def _sc_edge_pass(grp, htab, wtab, idxpk, zeros_tbl):
    mesh = plsc.VectorSubcoreMesh(core_axis_name="c", subcore_axis_name="s")
    fn = pl.kernel(
        _make_sc_body(grp),
        out_type=jax.ShapeDtypeStruct((grp, NC, ACC_N, G), jnp.float32),
        mesh=mesh,
        scratch_types=[
            pltpu.VMEM((NIR, 2, CH), jnp.int32),
            pltpu.VMEM((NRING, CH, G), jnp.float32),
            pltpu.VMEM((NRING, CH, G), jnp.float32),
            pltpu.VMEM_SHARED((ACC_N, G), jnp.float32),
            pltpu.SemaphoreType.DMA((NRING,)),
            pltpu.SemaphoreType.DMA((NRING,)),
            pltpu.SemaphoreType.DMA((NRING,)),
            pltpu.SemaphoreType.DMA((NIR,)),
        ],
    )
    return fn(htab, wtab, idxpk, zeros_tbl)


def _ec_body(agg_ref, wout_ref, lin2_ref, lin3_ref, win2_ref, h2_ref):
    outs = []
    for l in range(3):
        g, h = l // 2, l % 2
        a = (agg_ref[g, 0, :, h * C:(h + 1) * C]
             + agg_ref[g, 1, :, h * C:(h + 1) * C])
        o = jnp.dot(a, wout_ref[l], preferred_element_type=jnp.float32)
        nrm = jnp.sqrt(jnp.sum(o * o, axis=1, keepdims=True))
        o = o / (nrm + 1e-8)
        o = jnp.dot(o, lin2_ref[l], preferred_element_type=jnp.float32)
        o = _ssp(o)
        o = jnp.dot(o, lin3_ref[l], preferred_element_type=jnp.float32)
        outs.append(o)
    for k, (i, _f, _o) in enumerate(_COMBOS):
        g, h = k // 2, k % 2
        h2_ref[g, :, h * C:(h + 1) * C] = jnp.dot(
            outs[i], win2_ref[k], preferred_element_type=jnp.float32)
    h2_ref[7, :, C:2 * C] = jnp.zeros((NBLK, C), jnp.float32)


def _ec(agg1, woutS, lin2S, lin3S, win2S):
    return pl.pallas_call(
        _ec_body,
        grid=(N // NBLK,),
        in_specs=[
            pl.BlockSpec((2, NC, NBLK, G), lambda nb: (0, 0, nb, 0)),
            pl.BlockSpec((3, C, C), lambda nb: (0, 0, 0)),
            pl.BlockSpec((3, C, C), lambda nb: (0, 0, 0)),
            pl.BlockSpec((3, C, C), lambda nb: (0, 0, 0)),
            pl.BlockSpec((15, C, C), lambda nb: (0, 0, 0)),
        ],
        out_specs=pl.BlockSpec((8, NBLK, G), lambda nb: (0, nb, 0)),
        out_shape=jax.ShapeDtypeStruct((8, N, G), jnp.float32),
    )(agg1, woutS, lin2S, lin3S, win2S)


def _final_body(agg_ref, sel_ref,
                wout2_ref, lin40_ref, lin41_ref, lin42_ref,
                d1w_ref, d1b_ref, d2w_ref, d2b_ref, d3w_ref, d3b_ref,
                out_ref, acc_smem):
    lin4 = {0: lin40_ref, 1: lin41_ref, 2: lin42_ref}
    acc = {0: jnp.zeros((NBLK, C), jnp.float32),
           1: jnp.zeros((NBLK, C), jnp.float32),
           2: jnp.zeros((NBLK, C), jnp.float32)}
    pos = {0: 0, 1: 0, 2: 0}
    for k, (_i, _f, o) in enumerate(_COMBOS):
        g, h = k // 2, k % 2
        a = (agg_ref[g, 0, :, h * C:(h + 1) * C]
             + agg_ref[g, 1, :, h * C:(h + 1) * C])
        co = jnp.dot(a, wout2_ref[k], preferred_element_type=jnp.float32)
        j = pos[o]
        pos[o] += 1
        acc[o] = acc[o] + jnp.dot(co, lin4[o][j * C:(j + 1) * C, :],
                                  preferred_element_type=jnp.float32)
    feat = _ssp(acc[0]) + _ssp(acc[1]) + _ssp(acc[2])
    h = _elu(jnp.dot(feat, d1w_ref[...], preferred_element_type=jnp.float32)
             + d1b_ref[...])
    h = _elu(jnp.dot(h, d2w_ref[...], preferred_element_type=jnp.float32)
             + d2b_ref[...])
    pred = (jnp.dot(h, d3w_ref[...], preferred_element_type=jnp.float32)
            + d3b_ref[...])
    m = sel_ref[...] != 0
    psum = jnp.sum(jnp.where(m, pred, 0.0))
    pcnt = jnp.sum(m.astype(jnp.float32))

    @pl.when(pl.program_id(0) == 0)
    def _init():
        acc_smem[0] = 0.0
        acc_smem[1] = 0.0
    acc_smem[0] += psum
    acc_smem[1] += pcnt

    @pl.when(pl.program_id(0) == N // NBLK - 1)
    def _fin():
        out_ref[...] = jnp.full((1, 1), acc_smem[0] / acc_smem[1], jnp.float32)


def _final(agg2, sel2d, wout2S, lin40, lin41, lin42,
           d1w, d1b, d2w, d2b, d3w, d3b):
    full = lambda shape: pl.BlockSpec(shape, lambda nb: (0,) * len(shape))
    return pl.pallas_call(
        _final_body,
        grid=(N // NBLK,),
        in_specs=[
            pl.BlockSpec((8, NC, NBLK, G), lambda nb: (0, 0, nb, 0)),
            pl.BlockSpec((NBLK, 1), lambda nb: (nb, 0)),
            full((15, C, C)),
            full((3 * C, C)),
            full((6 * C, C)),
            full((6 * C, C)),
            full((C, 250)),
            full((1, 250)),
            full((250, 150)),
            full((1, 150)),
            full((150, 1)),
            full((1, 1)),
        ],
        out_specs=pl.BlockSpec((1, 1), lambda nb: (0, 0)),
        out_shape=jax.ShapeDtypeStruct((1, 1), jnp.float32),
        scratch_shapes=[pltpu.SMEM((2,), jnp.float32)],
    )(agg2, sel2d, wout2S, lin40, lin41, lin42,
      d1w, d1b, d2w, d2b, d3w, d3b)


def _bd2(ws):
    z = jnp.zeros((24, G), jnp.float32)
    for j, w in enumerate(ws):
        z = z.at[12 * j:12 * (j + 1), 64 * j:64 * (j + 1)].set(w)
    return z


def _layer_stacks(ps, max_radius, n_basis, ng):
    mu = jnp.linspace(0.0, max_radius, n_basis)
    a = (1.0 / (mu[1] - mu[0])).reshape(1, 1)
    amu = (a[0, 0] * mu).reshape(1, n_basis)
    zw1 = jnp.zeros((n_basis, 12), jnp.float32)
    zb1 = jnp.zeros((12,), jnp.float32)
    zw2 = jnp.zeros((12, C), jnp.float32)
    zb2 = jnp.zeros((C,), jnp.float32)
    w1c = jnp.concatenate([p['w1'] if p is not None else zw1 for p in ps], 1)
    b1c = jnp.concatenate(
        [p['b1'] if p is not None else zb1 for p in ps]).reshape(1, -1)
    bdS = jnp.zeros((24 * ng, G * ng), jnp.float32)
    for j in range(2 * ng):
        w2 = ps[j]['w2'] if ps[j] is not None else zw2
        bdS = bdS.at[12 * j:12 * (j + 1), C * j:C * (j + 1)].set(w2)
    b2S = jnp.stack([
        jnp.concatenate(
            [(ps[2 * g]['b2'] if ps[2 * g] is not None else zb2),
             (ps[2 * g + 1]['b2'] if ps[2 * g + 1] is not None else zb2)])
        for g in range(ng)])[:, None, :]
    return a, amu, w1c, b1c, bdS, b2S


def kernel(x, edge_index, edge_attr, select_ca, params):
    src = edge_index[0].astype(jnp.int32)
    dst = edge_index[1].astype(jnp.int32)
    pad = EP - E
    padidx = (jnp.arange(pad, dtype=jnp.int32) * 997) % N
    src_p = jnp.concatenate([src, padidx])
    dst_p = jnp.concatenate([dst, padidx])
    idxpk = jnp.stack([src_p.reshape(EP // CH, CH),
                       dst_p.reshape(EP // CH, CH)], axis=1)
    ea_p = jnp.concatenate(
        [edge_attr, jnp.zeros((pad, 3), jnp.float32)], axis=0)
    zeros_tbl = jnp.zeros((ACC_N, G), jnp.float32)

    st1 = _layer_stacks(
        [params['conv1_%d' % l] for l in range(3)] + [None], 10.0, 20, 2)
    st2 = _layer_stacks(
        [params['conv2_%d%d%d' % c] for c in _COMBOS] + [None], 20.0, 40, 8)
    wins1 = [params['conv1_%d' % l]['win'] for l in range(3)]
    wincat1 = jnp.stack([
        jnp.concatenate([wins1[0], wins1[1]], axis=1),
        jnp.concatenate([wins1[2], jnp.zeros((C, C), jnp.float32)], axis=1),
    ])
    woutS1 = jnp.stack([params['conv1_%d' % l]['wout'] for l in range(3)])
    lin2S = jnp.stack([params['lin2_%d' % l] for l in range(3)])
    lin3S = jnp.stack([params['lin3_%d' % l] for l in range(3)])
    win2S = jnp.stack([params['conv2_%d%d%d' % c]['win'] for c in _COMBOS])
    wout2S = jnp.stack([params['conv2_%d%d%d' % c]['wout'] for c in _COMBOS])

    W1 = _edge_weights(2, 20, ea_p, *st1)
    H1 = _node_h1(x, params['lin1'], wincat1)

    agg1 = _sc_edge_pass(2, H1.reshape(2 * N, G),
                         W1.reshape(2 * EP, G),
                         idxpk, zeros_tbl)
    W2 = _edge_weights(8, 40, ea_p, *st2)
    H2 = _ec(agg1, woutS1, lin2S, lin3S, win2S)
    agg2 = _sc_edge_pass(8, H2.reshape(8 * N, G),
                         W2.reshape(8 * EP, G),
                         idxpk, zeros_tbl)

    sel2d = select_ca.reshape(N, 1).astype(jnp.int32)
    out = _final(agg2, sel2d, wout2S,
                 params['lin40'], params['lin41'], params['lin42'],
                 params['d1w'], params['d1b'].reshape(1, 250),
                 params['d2w'], params['d2b'].reshape(1, 150),
                 params['d3w'], params['d3b'].reshape(1, 1))
    return out[0, 0]


# speedup vs baseline: 1.0275x; 1.0275x over previous
import jax
import jax.numpy as jnp
from jax import lax
from jax.experimental import pallas as pl
from jax.experimental.pallas import tpu as pltpu
from jax.experimental.pallas import tpu_sc as plsc

N = 10000
E = 320000
C = 64
G = 128
IN_DIM = 4

_COMBOS = [(i, f, o) for i in range(3) for f in range(3)
           for o in range(abs(f - i), min(i + f + 1, 3))]

NC = 2
NS = 16
NW = NC * NS
CH = 64
EP = 331776
CPW = EP // (CH * NW)
NRING = 3
NIR = 6

ACC_N = N
RPS = 632
RPS_LAST = ACC_N - 15 * RPS

EB = 2048
NBLK = 1000


def _ssp(x):
    return jnp.maximum(x, 0.0) + jnp.log1p(jnp.exp(-jnp.abs(x))) - jnp.log(2.0)


def _elu(x):
    return jnp.where(x > 0, x, jnp.exp(jnp.minimum(x, 0.0)) - 1.0)


def _make_edgew_body(ng):
    def _edgew_body(ea_ref, a_ref, amu_ref, w1_ref, b1_ref, bd_ref, b2_ref,
                    w_ref):
        ea = ea_ref[...]
        r = jnp.sqrt(jnp.sum(ea * ea, axis=1, keepdims=True) + 1e-12)
        sr = r * a_ref[0, 0]
        d = sr - amu_ref[...]
        rb = jnp.exp(-(d * d))
        a = jnp.maximum(
            jnp.dot(rb, w1_ref[...], preferred_element_type=jnp.float32)
            + b1_ref[...], 0.0)
        eb = pl.program_id(0)
        ids = eb * EB + lax.broadcasted_iota(jnp.int32, (EB, 1), 0)
        valid = ids < E
        for g in range(ng):
            w = (jnp.dot(a, bd_ref[:, G * g:G * (g + 1)],
                         preferred_element_type=jnp.float32) + b2_ref[g])
            w_ref[g] = jnp.where(valid, w, 0.0)
    return _edgew_body


def _edge_weights(ng, nb, ea_pad, aS, amuS, w1S, b1S, bdS, b2S):
    nblocks = EP // EB
    return pl.pallas_call(
        _make_edgew_body(ng),
        grid=(nblocks,),
        in_specs=[
            pl.BlockSpec((EB, 3), lambda eb: (eb, 0)),
            pl.BlockSpec((1, 1), lambda eb: (0, 0)),
            pl.BlockSpec((1, nb), lambda eb: (0, 0)),
            pl.BlockSpec((nb, 24 * ng), lambda eb: (0, 0)),
            pl.BlockSpec((1, 24 * ng), lambda eb: (0, 0)),
            pl.BlockSpec((24 * ng, G * ng), lambda eb: (0, 0)),
            pl.BlockSpec((ng, 1, G), lambda eb: (0, 0, 0)),
        ],
        out_specs=pl.BlockSpec((ng, EB, G), lambda eb: (0, eb, 0)),
        out_shape=jax.ShapeDtypeStruct((ng, EP, G), jnp.float32),
    )(ea_pad, aS, amuS, w1S, b1S, bdS, b2S)


def _nodeh1_body(x_ref, lin1_ref, win_ref, h_ref):
    out = jnp.dot(x_ref[...], lin1_ref[...], preferred_element_type=jnp.float32)
    for g in range(2):
        h_ref[g] = jnp.dot(out, win_ref[g], preferred_element_type=jnp.float32)


def _node_h1(x, lin1, wincat):
    return pl.pallas_call(
        _nodeh1_body,
        grid=(N // NBLK,),
        in_specs=[
            pl.BlockSpec((NBLK, IN_DIM), lambda nb: (nb, 0)),
            pl.BlockSpec((IN_DIM, C), lambda nb: (0, 0)),
            pl.BlockSpec((2, C, G), lambda nb: (0, 0, 0)),
        ],
        out_specs=pl.BlockSpec((2, NBLK, G), lambda nb: (0, nb, 0)),
        out_shape=jax.ShapeDtypeStruct((2, N, G), jnp.float32),
    )(x, lin1, wincat)


def _make_sc_body(grp):
    def _sc_body(htab, wtab, idxpk, zero_hbm, out_hbm,
                 idxb, rows, wv, acc_sh, gsem, wsem, ssem, isem):
        c = lax.axis_index("c")
        s = lax.axis_index("s")
        wid = s * NC + c
        r0 = s * RPS

        def group_body(g, carry):
            @pl.when(s < NS - 1)
            def _zero_main():
                pltpu.sync_copy(zero_hbm.at[pl.ds(r0, RPS)],
                                acc_sh.at[pl.ds(r0, RPS)])

            @pl.when(s == NS - 1)
            def _zero_tail():
                pltpu.sync_copy(zero_hbm.at[pl.ds(15 * RPS, RPS_LAST)],
                                acc_sh.at[pl.ds(15 * RPS, RPS_LAST)])
            plsc.subcore_barrier()
            off = g * N

            def issue_idx(ch):
                ib = ch % NIR
                pltpu.async_copy(idxpk.at[wid * CPW + ch], idxb.at[ib],
                                 isem.at[ib])

            def issue(ch, b):
                ib = ch % NIR
                gc = wid * CPW + ch
                pltpu.make_async_copy(idxpk.at[gc], idxb.at[ib],
                                      isem.at[ib]).wait()
                for j in range(CH // 16):
                    sl = pl.ds(j * 16, 16)
                    idxb[ib, 0, sl] = idxb[ib, 0, sl] + off
                pltpu.async_copy(htab.at[idxb.at[ib, 0]], rows.at[b],
                                 gsem.at[b])
                pltpu.async_copy(wtab.at[pl.ds(g * EP + gc * CH, CH)],
                                 wv.at[b], wsem.at[b])

            for p in range(4):
                issue_idx(p)
            issue(0, 0)
            issue(1, 1)

            def outer(io, carry2):
                for b in range(NRING):
                    ch = io * NRING + b
                    gc = wid * CPW + ch
                    ib = ch % NIR
                    pltpu.make_async_copy(htab.at[idxb.at[ib, 0]],
                                          rows.at[b], gsem.at[b]).wait()
                    pltpu.make_async_copy(wtab.at[pl.ds(g * EP + gc * CH, CH)],
                                          wv.at[b], wsem.at[b]).wait()

                    rb = rows.at[b]
                    wb = wv.at[b]

                    @plsc.parallel_loop(0, CH, step=1, unroll=4)
                    def mul_body(k):
                        sls = [pl.ds(cc * 16, 16) for cc in range(G // 16)]
                        rvals = [rb[k, sl] for sl in sls]
                        wvals = [wb[k, sl] for sl in sls]
                        for sl, rv, wv_ in zip(sls, rvals, wvals):
                            rb[k, sl] = rv * wv_

                    bp = (b + 2) % NRING

                    @pl.when(ch >= 1)
                    def _wait_prev_scatter():
                        pltpu.make_async_copy(
                            rows.at[bp], acc_sh.at[idxb.at[(ch + 5) % NIR, 1]],
                            ssem.at[bp]).wait()
                    pltpu.async_copy(rows.at[b], acc_sh.at[idxb.at[ib, 1]],
                                     ssem.at[b], add=True)

                    @pl.when(ch + 2 < CPW)
                    def _issue_ahead():
                        issue(ch + 2, (b + 2) % NRING)

                    @pl.when(ch + 4 < CPW)
                    def _issue_idx_ahead():
                        issue_idx(ch + 4)
                return carry2
            lax.fori_loop(0, CPW // NRING, outer, 0)

            bl = (CPW - 1) % NRING
            pltpu.make_async_copy(rows.at[bl],
                                  acc_sh.at[idxb.at[(CPW - 1) % NIR, 1]],
                                  ssem.at[bl]).wait()
            plsc.subcore_barrier()

            @pl.when(s < NS - 1)
            def _dump_main():
                pltpu.sync_copy(acc_sh.at[pl.ds(r0, RPS)],
                                out_hbm.at[g].at[c].at[pl.ds(r0, RPS)])

            @pl.when(s == NS - 1)
            def _dump_tail():
                pltpu.sync_copy(
                    acc_sh.at[pl.ds(15 * RPS, RPS_LAST)],
                    out_hbm.at[g].at[c].at[pl.ds(15 * RPS, RPS_LAST)])
            plsc.subcore_barrier()
            return carry
        lax.fori_loop(0, grp, group_body, 0)
    return _sc_body


def _sc_edge_pass(grp, htab, wtab, idxpk, zeros_tbl):
    mesh = plsc.VectorSubcoreMesh(core_axis_name="c", subcore_axis_name="s")
    fn = pl.kernel(
        _make_sc_body(grp),
        out_type=jax.ShapeDtypeStruct((grp, NC, ACC_N, G), jnp.float32),
        mesh=mesh,
        scratch_types=[
            pltpu.VMEM((NIR, 2, CH), jnp.int32),
            pltpu.VMEM((NRING, CH, G), jnp.float32),
            pltpu.VMEM((NRING, CH, G), jnp.float32),
            pltpu.VMEM_SHARED((ACC_N, G), jnp.float32),
            pltpu.SemaphoreType.DMA((NRING,)),
            pltpu.SemaphoreType.DMA((NRING,)),
            pltpu.SemaphoreType.DMA((NRING,)),
            pltpu.SemaphoreType.DMA((NIR,)),
        ],
    )
    return fn(htab, wtab, idxpk, zeros_tbl)


def _ec_body(agg_ref, wout_ref, lin2_ref, lin3_ref, win2_ref, h2_ref):
    outs = []
    for l in range(3):
        g, h = l // 2, l % 2
        a = (agg_ref[g, 0, :, h * C:(h + 1) * C]
             + agg_ref[g, 1, :, h * C:(h + 1) * C])
        o = jnp.dot(a, wout_ref[l], preferred_element_type=jnp.float32)
        nrm = jnp.sqrt(jnp.sum(o * o, axis=1, keepdims=True))
        o = o / (nrm + 1e-8)
        o = jnp.dot(o, lin2_ref[l], preferred_element_type=jnp.float32)
        o = _ssp(o)
        o = jnp.dot(o, lin3_ref[l], preferred_element_type=jnp.float32)
        outs.append(o)
    for k, (i, _f, _o) in enumerate(_COMBOS):
        g, h = k // 2, k % 2
        h2_ref[g, :, h * C:(h + 1) * C] = jnp.dot(
            outs[i], win2_ref[k], preferred_element_type=jnp.float32)
    h2_ref[7, :, C:2 * C] = jnp.zeros((NBLK, C), jnp.float32)


def _ec(agg1, woutS, lin2S, lin3S, win2S):
    return pl.pallas_call(
        _ec_body,
        grid=(N // NBLK,),
        in_specs=[
            pl.BlockSpec((2, NC, NBLK, G), lambda nb: (0, 0, nb, 0)),
            pl.BlockSpec((3, C, C), lambda nb: (0, 0, 0)),
            pl.BlockSpec((3, C, C), lambda nb: (0, 0, 0)),
            pl.BlockSpec((3, C, C), lambda nb: (0, 0, 0)),
            pl.BlockSpec((15, C, C), lambda nb: (0, 0, 0)),
        ],
        out_specs=pl.BlockSpec((8, NBLK, G), lambda nb: (0, nb, 0)),
        out_shape=jax.ShapeDtypeStruct((8, N, G), jnp.float32),
    )(agg1, woutS, lin2S, lin3S, win2S)


def _final_body(agg_ref, sel_ref,
                wout2_ref, lin40_ref, lin41_ref, lin42_ref,
                d1w_ref, d1b_ref, d2w_ref, d2b_ref, d3w_ref, d3b_ref,
                out_ref, acc_smem):
    lin4 = {0: lin40_ref, 1: lin41_ref, 2: lin42_ref}
    acc = {0: jnp.zeros((NBLK, C), jnp.float32),
           1: jnp.zeros((NBLK, C), jnp.float32),
           2: jnp.zeros((NBLK, C), jnp.float32)}
    pos = {0: 0, 1: 0, 2: 0}
    for k, (_i, _f, o) in enumerate(_COMBOS):
        g, h = k // 2, k % 2
        a = (agg_ref[g, 0, :, h * C:(h + 1) * C]
             + agg_ref[g, 1, :, h * C:(h + 1) * C])
        co = jnp.dot(a, wout2_ref[k], preferred_element_type=jnp.float32)
        j = pos[o]
        pos[o] += 1
        acc[o] = acc[o] + jnp.dot(co, lin4[o][j * C:(j + 1) * C, :],
                                  preferred_element_type=jnp.float32)
    feat = _ssp(acc[0]) + _ssp(acc[1]) + _ssp(acc[2])
    h = _elu(jnp.dot(feat, d1w_ref[...], preferred_element_type=jnp.float32)
             + d1b_ref[...])
    h = _elu(jnp.dot(h, d2w_ref[...], preferred_element_type=jnp.float32)
             + d2b_ref[...])
    pred = (jnp.dot(h, d3w_ref[...], preferred_element_type=jnp.float32)
            + d3b_ref[...])
    m = sel_ref[...] != 0
    psum = jnp.sum(jnp.where(m, pred, 0.0))
    pcnt = jnp.sum(m.astype(jnp.float32))

    @pl.when(pl.program_id(0) == 0)
    def _init():
        acc_smem[0] = 0.0
        acc_smem[1] = 0.0
    acc_smem[0] += psum
    acc_smem[1] += pcnt

    @pl.when(pl.program_id(0) == N // NBLK - 1)
    def _fin():
        out_ref[...] = jnp.full((1, 1), acc_smem[0] / acc_smem[1], jnp.float32)


def _final(agg2, sel2d, wout2S, lin40, lin41, lin42,
           d1w, d1b, d2w, d2b, d3w, d3b):
    full = lambda shape: pl.BlockSpec(shape, lambda nb: (0,) * len(shape))
    return pl.pallas_call(
        _final_body,
        grid=(N // NBLK,),
        in_specs=[
            pl.BlockSpec((8, NC, NBLK, G), lambda nb: (0, 0, nb, 0)),
            pl.BlockSpec((NBLK, 1), lambda nb: (nb, 0)),
            full((15, C, C)),
            full((3 * C, C)),
            full((6 * C, C)),
            full((6 * C, C)),
            full((C, 250)),
            full((1, 250)),
            full((250, 150)),
            full((1, 150)),
            full((150, 1)),
            full((1, 1)),
        ],
        out_specs=pl.BlockSpec((1, 1), lambda nb: (0, 0)),
        out_shape=jax.ShapeDtypeStruct((1, 1), jnp.float32),
        scratch_shapes=[pltpu.SMEM((2,), jnp.float32)],
    )(agg2, sel2d, wout2S, lin40, lin41, lin42,
      d1w, d1b, d2w, d2b, d3w, d3b)


def _bd2(ws):
    z = jnp.zeros((24, G), jnp.float32)
    for j, w in enumerate(ws):
        z = z.at[12 * j:12 * (j + 1), 64 * j:64 * (j + 1)].set(w)
    return z


def _layer_stacks(ps, max_radius, n_basis, ng):
    mu = jnp.linspace(0.0, max_radius, n_basis)
    a = (1.0 / (mu[1] - mu[0])).reshape(1, 1)
    amu = (a[0, 0] * mu).reshape(1, n_basis)
    zw1 = jnp.zeros((n_basis, 12), jnp.float32)
    zb1 = jnp.zeros((12,), jnp.float32)
    zw2 = jnp.zeros((12, C), jnp.float32)
    zb2 = jnp.zeros((C,), jnp.float32)
    w1c = jnp.concatenate([p['w1'] if p is not None else zw1 for p in ps], 1)
    b1c = jnp.concatenate(
        [p['b1'] if p is not None else zb1 for p in ps]).reshape(1, -1)
    bdS = jnp.zeros((24 * ng, G * ng), jnp.float32)
    for j in range(2 * ng):
        w2 = ps[j]['w2'] if ps[j] is not None else zw2
        bdS = bdS.at[12 * j:12 * (j + 1), C * j:C * (j + 1)].set(w2)
    b2S = jnp.stack([
        jnp.concatenate(
            [(ps[2 * g]['b2'] if ps[2 * g] is not None else zb2),
             (ps[2 * g + 1]['b2'] if ps[2 * g + 1] is not None else zb2)])
        for g in range(ng)])[:, None, :]
    return a, amu, w1c, b1c, bdS, b2S


def kernel(x, edge_index, edge_attr, select_ca, params):
    src = edge_index[0].astype(jnp.int32)
    dst = edge_index[1].astype(jnp.int32)
    pad = EP - E
    padidx = (jnp.arange(pad, dtype=jnp.int32) * 997) % N
    src_p = jnp.concatenate([src, padidx])
    dst_p = jnp.concatenate([dst, padidx])
    idxpk = jnp.stack([src_p.reshape(EP // CH, CH),
                       dst_p.reshape(EP // CH, CH)], axis=1)
    ea_p = jnp.concatenate(
        [edge_attr, jnp.zeros((pad, 3), jnp.float32)], axis=0)
    zeros_tbl = jnp.zeros((ACC_N, G), jnp.float32)

    st1 = _layer_stacks(
        [params['conv1_%d' % l] for l in range(3)] + [None], 10.0, 20, 2)
    st2 = _layer_stacks(
        [params['conv2_%d%d%d' % c] for c in _COMBOS] + [None], 20.0, 40, 8)
    wins1 = [params['conv1_%d' % l]['win'] for l in range(3)]
    wincat1 = jnp.stack([
        jnp.concatenate([wins1[0], wins1[1]], axis=1),
        jnp.concatenate([wins1[2], jnp.zeros((C, C), jnp.float32)], axis=1),
    ])
    woutS1 = jnp.stack([params['conv1_%d' % l]['wout'] for l in range(3)])
    lin2S = jnp.stack([params['lin2_%d' % l] for l in range(3)])
    lin3S = jnp.stack([params['lin3_%d' % l] for l in range(3)])
    win2S = jnp.stack([params['conv2_%d%d%d' % c]['win'] for c in _COMBOS])
    wout2S = jnp.stack([params['conv2_%d%d%d' % c]['wout'] for c in _COMBOS])

    W1 = _edge_weights(2, 20, ea_p, *st1)
    H1 = _node_h1(x, params['lin1'], wincat1)

    W2 = _edge_weights(8, 40, ea_p, *st2)

    agg1 = _sc_edge_pass(2, H1.reshape(2 * N, G),
                         W1.reshape(2 * EP, G),
                         idxpk, zeros_tbl)
    H2 = _ec(agg1, woutS1, lin2S, lin3S, win2S)
    agg2 = _sc_edge_pass(8, H2.reshape(8 * N, G),
                         W2.reshape(8 * EP, G),
                         idxpk, zeros_tbl)

    sel2d = select_ca.reshape(N, 1).astype(jnp.int32)
    out = _final(agg2, sel2d, wout2S,
                 params['lin40'], params['lin41'], params['lin42'],
                 params['d1w'], params['d1b'].reshape(1, 250),
                 params['d2w'], params['d2b'].reshape(1, 150),
                 params['d3w'], params['d3b'].reshape(1, 1))
    return out[0, 0]
